# Initial kernel scaffold; baseline (speedup 1.0000x reference)
import jax
import jax.numpy as jnp
from jax.experimental import pallas as pl


def kernel(x, RW1, Rb1, RW2, Rb2, EW0, Eb0, EW1, Eb1, EW2, Eb2, EW3, Eb3, EW4, Eb4, EW5, Eb5):
    raise NotImplementedError("write your pallas kernel here")


# speedup vs baseline: 2.6743x; 2.6743x over previous
import jax
import jax.numpy as jnp
import numpy as np
from jax.experimental import pallas as pl

NUM_ENC = 6
TEMP = 0.1
OMEGAS = (50.0, 60.0, 60.0, 70.0, 70.0, 50.0)
TILE = 1024

_SIN_C = (0.9999996708398818, -0.1666656814169894, 0.00833249623126527,
          -0.00019810890642008838, 2.7020794812703656e-06,
          -2.0451534191690273e-08)
_INV2PI = 0.15915494309189535
_RC1 = 6.28125
_RC2 = 0.0019353071795864769


def _fast_sin(z):
    k = jnp.round(z * _INV2PI)
    r = (z - k * _RC1) - k * _RC2
    r2 = r * r
    p = _SIN_C[5]
    for i in range(4, -1, -1):
        p = p * r2 + _SIN_C[i]
    return p * r


def _fused_kernel(x_ref, RW1_ref, Rb1_ref, RW2_ref, Rb2_ref,
                  W0_ref, b0_ref, W1_ref, b1_ref, W2_ref, b2_ref,
                  W3_ref, b3_ref, W4_ref, b4_ref, W5_ref, b5_ref,
                  out_ref):
    x = x_ref[...]
    parts = [x]
    for i in range(NUM_ENC):
        s = (2.0 ** i) * np.pi
        parts.append(jnp.sin(s * x))
        parts.append(jnp.cos(s * x))
    enc = jnp.concatenate(parts, axis=-1)

    hr = jnp.dot(enc, RW1_ref[...], preferred_element_type=jnp.float32)
    hr = jnp.maximum(hr + Rb1_ref[...], 0.0)
    logits = jnp.dot(hr, RW2_ref[...], preferred_element_type=jnp.float32)
    logits = (logits + Rb2_ref[...]) * (1.0 / TEMP)
    m = jnp.max(logits, axis=-1, keepdims=True)
    ex = jnp.exp(logits - m)
    w = ex / jnp.sum(ex, axis=-1, keepdims=True)

    z0 = jnp.dot(enc, W0_ref[...], preferred_element_type=jnp.float32)
    h0 = _fast_sin(OMEGAS[0] * (z0 + b0_ref[...]))
    hs = [h0[:, 256 * e:256 * (e + 1)] for e in range(6)]

    for li, W_ref, b_ref in ((1, W1_ref, b1_ref), (2, W2_ref, b2_ref),
                             (3, W3_ref, b3_ref), (4, W4_ref, b4_ref)):
        zs = [jnp.dot(hs[e], W_ref[e], preferred_element_type=jnp.float32)
              for e in range(6)]
        hs = [_fast_sin(OMEGAS[li] * (zs[e] + b_ref[e])) for e in range(6)]

    acc = jnp.zeros((x.shape[0], 3), jnp.float32)
    for e in range(6):
        z = jnp.dot(hs[e], W5_ref[e], preferred_element_type=jnp.float32)
        y = _fast_sin(OMEGAS[5] * (z + b5_ref[e]))
        acc = acc + w[:, e:e + 1] * y
    out_ref[...] = acc


def kernel(x, RW1, Rb1, RW2, Rb2, EW0, Eb0, EW1, Eb1, EW2, Eb2,
           EW3, Eb3, EW4, Eb4, EW5, Eb5):
    n = x.shape[0]
    tile = TILE if n % TILE == 0 else n
    grid = n // tile

    Rb1_2d = Rb1.reshape(1, -1)
    RW2_s = RW2
    Rb2_2d = Rb2.reshape(1, -1)
    W0c = EW0.transpose(1, 0, 2).reshape(EW0.shape[1], -1)
    b0c = Eb0.reshape(1, -1)
    Wf = [EW1, EW2, EW3, EW4, EW5]
    bf = [Eb.reshape(Eb.shape[0], 1, Eb.shape[1])
          for Eb in (Eb1, Eb2, Eb3, Eb4, Eb5)]

    def full(a):
        nd = a.ndim
        return pl.BlockSpec(a.shape, lambda i, _nd=nd: (0,) * _nd)

    operands = (RW1, Rb1_2d, RW2_s, Rb2_2d, W0c, b0c,
                Wf[0], bf[0], Wf[1], bf[1], Wf[2], bf[2],
                Wf[3], bf[3], Wf[4], bf[4])
    in_specs = [pl.BlockSpec((tile, 2), lambda i: (i, 0))]
    in_specs += [full(a) for a in operands]

    out = pl.pallas_call(
        _fused_kernel,
        grid=(grid,),
        in_specs=in_specs,
        out_specs=pl.BlockSpec((tile, 3), lambda i: (i, 0)),
        out_shape=jax.ShapeDtypeStruct((n, 3), jnp.float32),
    )(x, *operands)
    return out


# speedup vs baseline: 3.2022x; 1.1974x over previous
import jax
import jax.numpy as jnp
import numpy as np
from jax.experimental import pallas as pl

NUM_ENC = 6
TEMP = 0.1
OMEGAS = (50.0, 60.0, 60.0, 70.0, 70.0, 50.0)
TILE = 512

_SIN_C = (0.9999996708398818, -0.1666656814169894, 0.00833249623126527,
          -0.00019810890642008838, 2.7020794812703656e-06,
          -2.0451534191690273e-08)
_INV2PI = 0.15915494309189535
_RC1 = 6.28125
_RC2 = 0.0019353071795864769


def _fast_sin(z):
    k = jnp.round(z * _INV2PI)
    r = (z - k * _RC1) - k * _RC2
    r2 = r * r
    p = _SIN_C[5]
    for i in range(4, -1, -1):
        p = p * r2 + _SIN_C[i]
    return p * r


def _fused_kernel(x_ref, RW1_ref, Rb1_ref, RW2_ref, Rb2_ref,
                  W0_ref, b0_ref, W1_ref, b1_ref, W2_ref, b2_ref,
                  W3_ref, b3_ref, W4_ref, b4_ref, W5_ref, b5_ref,
                  out_ref):
    x = x_ref[...]
    parts = [x]
    for i in range(NUM_ENC):
        s = (2.0 ** i) * np.pi
        parts.append(jnp.sin(s * x))
        parts.append(jnp.cos(s * x))
    enc = jnp.concatenate(parts, axis=-1)

    hr = jnp.dot(enc, RW1_ref[...], preferred_element_type=jnp.float32)
    hr = jnp.maximum(hr + Rb1_ref[...], 0.0)
    logits = jnp.dot(hr, RW2_ref[...], preferred_element_type=jnp.float32)
    logits = (logits + Rb2_ref[...]) * (1.0 / TEMP)
    m = jnp.max(logits, axis=-1, keepdims=True)
    ex = jnp.exp(logits - m)
    w = ex / jnp.sum(ex, axis=-1, keepdims=True)

    z0 = jnp.dot(enc, W0_ref[...], preferred_element_type=jnp.float32)
    h0 = _fast_sin(OMEGAS[0] * (z0 + b0_ref[...]))
    hs = [h0[:, 256 * e:256 * (e + 1)] for e in range(6)]

    for li, W_ref, b_ref in ((1, W1_ref, b1_ref), (2, W2_ref, b2_ref),
                             (3, W3_ref, b3_ref), (4, W4_ref, b4_ref)):
        zs = [jnp.dot(hs[e], W_ref[e], preferred_element_type=jnp.float32)
              for e in range(6)]
        hs = [_fast_sin(OMEGAS[li] * (zs[e] + b_ref[e])) for e in range(6)]

    acc = jnp.zeros((x.shape[0], 3), jnp.float32)
    for e in range(6):
        z = jnp.dot(hs[e], W5_ref[e], preferred_element_type=jnp.float32)
        y = _fast_sin(OMEGAS[5] * (z + b5_ref[e]))
        acc = acc + w[:, e:e + 1] * y
    out_ref[...] = acc


def kernel(x, RW1, Rb1, RW2, Rb2, EW0, Eb0, EW1, Eb1, EW2, Eb2,
           EW3, Eb3, EW4, Eb4, EW5, Eb5):
    n = x.shape[0]
    tile = TILE if n % TILE == 0 else n
    grid = n // tile

    Rb1_2d = Rb1.reshape(1, -1)
    RW2_s = RW2
    Rb2_2d = Rb2.reshape(1, -1)
    W0c = EW0.transpose(1, 0, 2).reshape(EW0.shape[1], -1)
    b0c = Eb0.reshape(1, -1)
    Wf = [EW1, EW2, EW3, EW4, EW5]
    bf = [Eb.reshape(Eb.shape[0], 1, Eb.shape[1])
          for Eb in (Eb1, Eb2, Eb3, Eb4, Eb5)]

    def full(a):
        nd = a.ndim
        return pl.BlockSpec(a.shape, lambda i, _nd=nd: (0,) * _nd)

    operands = (RW1, Rb1_2d, RW2_s, Rb2_2d, W0c, b0c,
                Wf[0], bf[0], Wf[1], bf[1], Wf[2], bf[2],
                Wf[3], bf[3], Wf[4], bf[4])
    in_specs = [pl.BlockSpec((tile, 2), lambda i: (i, 0))]
    in_specs += [full(a) for a in operands]

    out = pl.pallas_call(
        _fused_kernel,
        grid=(grid,),
        in_specs=in_specs,
        out_specs=pl.BlockSpec((tile, 3), lambda i: (i, 0)),
        out_shape=jax.ShapeDtypeStruct((n, 3), jnp.float32),
    )(x, *operands)
    return out
